# baseline (speedup 1.0000x reference)
import functools

import jax
import jax.numpy as jnp
from jax import lax
from jax.experimental import pallas as pl
from jax.experimental.pallas import tpu as pltpu
from jax.experimental.pallas import tpu_sc as plsc

NC = 2
NS = 16
LW = 128
CE = 128
NRND = 2


def _segment_sum_sc(n_slices, n_nodes, ngrp):
  n_passes = n_slices // NC
  n_acc = -(-(n_nodes + 1) // LW) * LW
  rpt = n_acc // NS
  nr = ngrp // NRND
  mesh = plsc.VectorSubcoreMesh(
      core_axis_name="c", subcore_axis_name="s", num_cores=NC)

  @functools.partial(
      pl.kernel,
      out_type=jax.ShapeDtypeStruct((n_slices, n_acc, LW), jnp.float32),
      mesh=mesh,
      scratch_types=[
          pltpu.VMEM((nr, CE), jnp.int32),
          pltpu.VMEM((nr, CE), jnp.int32),
          pltpu.VMEM((CE, LW), jnp.float32),
          pltpu.VMEM_SHARED((n_acc, LW), jnp.float32),
          pltpu.SemaphoreType.DMA,
      ],
  )
  def k(table, gidx, dst3, zrows, out, dst_v, gidx_v, rows_v, acc, sem):
    c = lax.axis_index("c")
    s = lax.axis_index("s")
    for t in range(n_passes):
      sl = c * n_passes + t
      pltpu.sync_copy(zrows.at[pl.ds(s * rpt, rpt)],
                      acc.at[pl.ds(s * rpt, rpt)])
      plsc.subcore_barrier()
      for r in range(NRND):
        pltpu.sync_copy(dst3.at[s, r], dst_v)
        pltpu.sync_copy(gidx.at[sl, s, r], gidx_v)
        def grp(j, carry):
          pltpu.async_copy(table.at[gidx_v.at[j]], rows_v, sem).wait()
          pltpu.sync_copy(rows_v, acc.at[dst_v.at[j]], add=True)
          return carry

        lax.fori_loop(0, nr, grp, 0)
      plsc.subcore_barrier()
      pltpu.sync_copy(acc.at[pl.ds(s * rpt, rpt)],
                      out.at[sl, pl.ds(s * rpt, rpt)])
      plsc.subcore_barrier()

  return k


def _mlp_tc(x, agg, w1, b1, w2, bm):
  m, k = x.shape
  kh = w1.shape[1]
  n = w2.shape[1]

  def body(x_ref, a_ref, w1_ref, b1_ref, w2_ref, o_ref):
    h = jnp.dot(x_ref[...] + a_ref[...], w1_ref[...],
                preferred_element_type=jnp.float32)
    h = jnp.maximum(h + b1_ref[...], 0.0)
    o_ref[...] = jnp.dot(h, w2_ref[...], preferred_element_type=jnp.float32)

  return pl.pallas_call(
      body,
      grid=(m // bm,),
      in_specs=[
          pl.BlockSpec((bm, k), lambda i: (i, 0)),
          pl.BlockSpec((bm, k), lambda i: (i, 0)),
          pl.BlockSpec((k, kh), lambda i: (0, 0)),
          pl.BlockSpec((1, kh), lambda i: (0, 0)),
          pl.BlockSpec((kh, n), lambda i: (0, 0)),
      ],
      out_specs=pl.BlockSpec((bm, n), lambda i: (i, 0)),
      out_shape=jax.ShapeDtypeStruct((m, n), jnp.float32),
  )(x, agg, w1, b1.reshape(1, kh), w2)


def _add_norm_tc(p, agg, b, bm):
  m, n = p.shape

  def body(p_ref, a_ref, b_ref, o_ref):
    acc = p_ref[...] + a_ref[...] + b_ref[...]
    nrm = jnp.sqrt(jnp.sum(acc * acc, axis=-1, keepdims=True))
    o_ref[...] = acc / jnp.maximum(nrm, 1e-12)

  return pl.pallas_call(
      body,
      grid=(m // bm,),
      in_specs=[
          pl.BlockSpec((bm, n), lambda i: (i, 0)),
          pl.BlockSpec((bm, n), lambda i: (i, 0)),
          pl.BlockSpec((1, n), lambda i: (0, 0)),
      ],
      out_specs=pl.BlockSpec((bm, n), lambda i: (i, 0)),
      out_shape=jax.ShapeDtypeStruct((m, n), jnp.float32),
  )(p, agg, b.reshape(1, n))


def kernel(x, edge_index, W1, b1, W2, b2):
  n_nodes, d_in = x.shape
  d_out = W2.shape[1]
  e = edge_index.shape[1]
  s1 = d_in // LW
  s2 = d_out // LW
  n_acc = -(-(n_nodes + 1) // LW) * LW

  ept = -(-e // NS)
  ept = -(-ept // (2 * CE * NRND)) * (2 * CE * NRND)
  ngrp = ept // CE
  e_pad = ept * NS
  src = edge_index[0]
  dst = edge_index[1]
  pad = e_pad - e
  trash = n_nodes + (jnp.arange(pad, dtype=jnp.int32) % (n_acc - n_nodes))
  src_p = jnp.concatenate([src, jnp.zeros((pad,), jnp.int32)])
  dst_p = jnp.concatenate([dst, trash])
  nr = ngrp // NRND
  dst3 = dst_p.reshape(NS, NRND, nr, CE)
  offs1 = (jnp.arange(s1, dtype=jnp.int32) * n_nodes)[:, None]
  gidx1 = (src_p[None, :] + offs1).reshape(s1, NS, NRND, nr, CE)
  offs2 = (jnp.arange(s2, dtype=jnp.int32) * n_nodes)[:, None]
  gidx2 = (src_p[None, :] + offs2).reshape(s2, NS, NRND, nr, CE)
  zrows = jnp.zeros((n_acc, LW), jnp.float32)

  table1 = x.reshape(n_nodes, s1, LW).transpose(1, 0, 2).reshape(s1 * n_nodes, LW)
  agg1_sl = _segment_sum_sc(s1, n_nodes, ngrp)(table1, gidx1, dst3, zrows)
  agg1 = agg1_sl[:, :n_nodes, :].transpose(1, 0, 2).reshape(n_nodes, d_in)
  p = _mlp_tc(x, agg1, W1, b1, W2, bm=1000)

  table2 = p.reshape(n_nodes, s2, LW).transpose(1, 0, 2).reshape(s2 * n_nodes, LW)
  agg2_sl = _segment_sum_sc(s2, n_nodes, ngrp)(table2, gidx2, dst3, zrows)
  agg2 = agg2_sl[:, :n_nodes, :].transpose(1, 0, 2).reshape(n_nodes, d_out)
  out = _add_norm_tc(p, agg2, b2, bm=1000)
  return out


# speedup vs baseline: 1.0047x; 1.0047x over previous
import functools

import jax
import jax.numpy as jnp
from jax import lax
from jax.experimental import pallas as pl
from jax.experimental.pallas import tpu as pltpu
from jax.experimental.pallas import tpu_sc as plsc

NC = 2
NS = 16
LW = 128
CE = 128
NRND = 1


def _segment_sum_sc(n_slices, n_nodes, ngrp):
  n_passes = n_slices // NC
  n_acc = -(-(n_nodes + 1) // LW) * LW
  rpt = n_acc // NS
  nr = ngrp // NRND
  mesh = plsc.VectorSubcoreMesh(
      core_axis_name="c", subcore_axis_name="s", num_cores=NC)

  @functools.partial(
      pl.kernel,
      out_type=jax.ShapeDtypeStruct((n_slices, n_acc, LW), jnp.float32),
      mesh=mesh,
      scratch_types=[
          pltpu.VMEM((nr, CE), jnp.int32),
          pltpu.VMEM((nr, CE), jnp.int32),
          pltpu.VMEM((CE, LW), jnp.float32),
          pltpu.VMEM_SHARED((n_acc, LW), jnp.float32),
          pltpu.SemaphoreType.DMA,
      ],
  )
  def k(table, gidx, dst3, zrows, out, dst_v, gidx_v, rows_v, acc, sem):
    c = lax.axis_index("c")
    s = lax.axis_index("s")
    for t in range(n_passes):
      sl = c * n_passes + t
      pltpu.sync_copy(zrows.at[pl.ds(s * rpt, rpt)],
                      acc.at[pl.ds(s * rpt, rpt)])
      plsc.subcore_barrier()
      for r in range(NRND):
        pltpu.sync_copy(dst3.at[s, r], dst_v)
        pltpu.sync_copy(gidx.at[sl, s, r], gidx_v)
        def grp(j, carry):
          pltpu.async_copy(table.at[gidx_v.at[j]], rows_v, sem).wait()
          pltpu.sync_copy(rows_v, acc.at[dst_v.at[j]], add=True)
          return carry

        lax.fori_loop(0, nr, grp, 0)
      plsc.subcore_barrier()
      pltpu.sync_copy(acc.at[pl.ds(s * rpt, rpt)],
                      out.at[sl, pl.ds(s * rpt, rpt)])
      plsc.subcore_barrier()

  return k


def _mlp_tc(x, agg, w1, b1, w2, bm):
  m, k = x.shape
  kh = w1.shape[1]
  n = w2.shape[1]

  def body(x_ref, a_ref, w1_ref, b1_ref, w2_ref, o_ref):
    h = jnp.dot(x_ref[...] + a_ref[...], w1_ref[...],
                preferred_element_type=jnp.float32)
    h = jnp.maximum(h + b1_ref[...], 0.0)
    o_ref[...] = jnp.dot(h, w2_ref[...], preferred_element_type=jnp.float32)

  return pl.pallas_call(
      body,
      grid=(m // bm,),
      in_specs=[
          pl.BlockSpec((bm, k), lambda i: (i, 0)),
          pl.BlockSpec((bm, k), lambda i: (i, 0)),
          pl.BlockSpec((k, kh), lambda i: (0, 0)),
          pl.BlockSpec((1, kh), lambda i: (0, 0)),
          pl.BlockSpec((kh, n), lambda i: (0, 0)),
      ],
      out_specs=pl.BlockSpec((bm, n), lambda i: (i, 0)),
      out_shape=jax.ShapeDtypeStruct((m, n), jnp.float32),
  )(x, agg, w1, b1.reshape(1, kh), w2)


def _add_norm_tc(p, agg, b, bm):
  m, n = p.shape

  def body(p_ref, a_ref, b_ref, o_ref):
    acc = p_ref[...] + a_ref[...] + b_ref[...]
    nrm = jnp.sqrt(jnp.sum(acc * acc, axis=-1, keepdims=True))
    o_ref[...] = acc / jnp.maximum(nrm, 1e-12)

  return pl.pallas_call(
      body,
      grid=(m // bm,),
      in_specs=[
          pl.BlockSpec((bm, n), lambda i: (i, 0)),
          pl.BlockSpec((bm, n), lambda i: (i, 0)),
          pl.BlockSpec((1, n), lambda i: (0, 0)),
      ],
      out_specs=pl.BlockSpec((bm, n), lambda i: (i, 0)),
      out_shape=jax.ShapeDtypeStruct((m, n), jnp.float32),
  )(p, agg, b.reshape(1, n))


def kernel(x, edge_index, W1, b1, W2, b2):
  n_nodes, d_in = x.shape
  d_out = W2.shape[1]
  e = edge_index.shape[1]
  s1 = d_in // LW
  s2 = d_out // LW
  n_acc = -(-(n_nodes + 1) // LW) * LW

  ept = -(-e // NS)
  ept = -(-ept // (2 * CE * NRND)) * (2 * CE * NRND)
  ngrp = ept // CE
  e_pad = ept * NS
  src = edge_index[0]
  dst = edge_index[1]
  pad = e_pad - e
  trash = n_nodes + (jnp.arange(pad, dtype=jnp.int32) % (n_acc - n_nodes))
  src_p = jnp.concatenate([src, jnp.zeros((pad,), jnp.int32)])
  dst_p = jnp.concatenate([dst, trash])
  nr = ngrp // NRND
  dst3 = dst_p.reshape(NS, NRND, nr, CE)
  offs1 = (jnp.arange(s1, dtype=jnp.int32) * n_nodes)[:, None]
  gidx1 = (src_p[None, :] + offs1).reshape(s1, NS, NRND, nr, CE)
  offs2 = (jnp.arange(s2, dtype=jnp.int32) * n_nodes)[:, None]
  gidx2 = (src_p[None, :] + offs2).reshape(s2, NS, NRND, nr, CE)
  zrows = jnp.zeros((n_acc, LW), jnp.float32)

  table1 = x.reshape(n_nodes, s1, LW).transpose(1, 0, 2).reshape(s1 * n_nodes, LW)
  agg1_sl = _segment_sum_sc(s1, n_nodes, ngrp)(table1, gidx1, dst3, zrows)
  agg1 = agg1_sl[:, :n_nodes, :].transpose(1, 0, 2).reshape(n_nodes, d_in)
  p = _mlp_tc(x, agg1, W1, b1, W2, bm=1000)

  table2 = p.reshape(n_nodes, s2, LW).transpose(1, 0, 2).reshape(s2 * n_nodes, LW)
  agg2_sl = _segment_sum_sc(s2, n_nodes, ngrp)(table2, gidx2, dst3, zrows)
  agg2 = agg2_sl[:, :n_nodes, :].transpose(1, 0, 2).reshape(n_nodes, d_out)
  out = _add_norm_tc(p, agg2, b2, bm=1000)
  return out


# speedup vs baseline: 1.4022x; 1.3957x over previous
import functools

import jax
import jax.numpy as jnp
from jax import lax
from jax.experimental import pallas as pl
from jax.experimental.pallas import tpu as pltpu
from jax.experimental.pallas import tpu_sc as plsc

NC = 2
NS = 16
LW = 128
CE = 128
NRND = 1


def _segment_sum_sc(n_slices, n_nodes, nch):
  n_passes = n_slices // NC
  n_acc = -(-(n_nodes + 1) // LW) * LW
  rpt = n_acc // NS
  mesh = plsc.VectorSubcoreMesh(
      core_axis_name="c", subcore_axis_name="s", num_cores=NC)

  @functools.partial(
      pl.kernel,
      out_type=jax.ShapeDtypeStruct((n_slices, n_acc, LW), jnp.float32),
      mesh=mesh,
      scratch_types=[
          pltpu.VMEM((nch, CE), jnp.int32),
          pltpu.VMEM((nch, CE), jnp.int32),
          pltpu.VMEM((CE, LW), jnp.float32),
          pltpu.VMEM_SHARED((n_acc, LW), jnp.float32),
          pltpu.SemaphoreType.DMA,
      ],
  )
  def k(table, gidx, dst3, zrows, out, dst_v, gidx_v, rows_v, acc, sem):
    c = lax.axis_index("c")
    s = lax.axis_index("s")
    pltpu.sync_copy(dst3.at[s], dst_v)
    for t in range(n_passes):
      sl = c * n_passes + t
      pltpu.sync_copy(zrows.at[pl.ds(s * rpt, rpt)],
                      acc.at[pl.ds(s * rpt, rpt)])
      pltpu.sync_copy(gidx.at[sl, s], gidx_v)
      plsc.subcore_barrier()

      def chunk(j, carry):
        pltpu.async_copy(table.at[gidx_v.at[j]], rows_v, sem).wait()
        pltpu.sync_copy(rows_v, acc.at[dst_v.at[j]], add=True)
        return carry

      lax.fori_loop(0, nch, chunk, 0)
      plsc.subcore_barrier()
      pltpu.sync_copy(acc.at[pl.ds(s * rpt, rpt)],
                      out.at[sl, pl.ds(s * rpt, rpt)])
      plsc.subcore_barrier()

  return k


def _mlp_tc(x, agg, w1, b1, w2, bm):
  m, k = x.shape
  kh = w1.shape[1]
  n = w2.shape[1]

  def body(x_ref, a_ref, w1_ref, b1_ref, w2_ref, o_ref):
    h = jnp.dot(x_ref[...] + a_ref[...], w1_ref[...],
                preferred_element_type=jnp.float32)
    h = jnp.maximum(h + b1_ref[...], 0.0)
    o_ref[...] = jnp.dot(h, w2_ref[...], preferred_element_type=jnp.float32)

  return pl.pallas_call(
      body,
      grid=(m // bm,),
      in_specs=[
          pl.BlockSpec((bm, k), lambda i: (i, 0)),
          pl.BlockSpec((bm, k), lambda i: (i, 0)),
          pl.BlockSpec((k, kh), lambda i: (0, 0)),
          pl.BlockSpec((1, kh), lambda i: (0, 0)),
          pl.BlockSpec((kh, n), lambda i: (0, 0)),
      ],
      out_specs=pl.BlockSpec((bm, n), lambda i: (i, 0)),
      out_shape=jax.ShapeDtypeStruct((m, n), jnp.float32),
  )(x, agg, w1, b1.reshape(1, kh), w2)


def _add_norm_tc(p, agg, b, bm):
  m, n = p.shape

  def body(p_ref, a_ref, b_ref, o_ref):
    acc = p_ref[...] + a_ref[...] + b_ref[...]
    nrm = jnp.sqrt(jnp.sum(acc * acc, axis=-1, keepdims=True))
    o_ref[...] = acc / jnp.maximum(nrm, 1e-12)

  return pl.pallas_call(
      body,
      grid=(m // bm,),
      in_specs=[
          pl.BlockSpec((bm, n), lambda i: (i, 0)),
          pl.BlockSpec((bm, n), lambda i: (i, 0)),
          pl.BlockSpec((1, n), lambda i: (0, 0)),
      ],
      out_specs=pl.BlockSpec((bm, n), lambda i: (i, 0)),
      out_shape=jax.ShapeDtypeStruct((m, n), jnp.float32),
  )(p, agg, b.reshape(1, n))


def kernel(x, edge_index, W1, b1, W2, b2):
  n_nodes, d_in = x.shape
  d_out = W2.shape[1]
  e = edge_index.shape[1]
  s1 = d_in // LW
  s2 = d_out // LW
  n_acc = -(-(n_nodes + 1) // LW) * LW

  ept = -(-e // NS)
  ept = -(-ept // CE) * CE
  nch = ept // CE
  e_pad = ept * NS
  src = edge_index[0]
  dst = edge_index[1]
  pad = e_pad - e
  src_p = jnp.concatenate([src, jnp.zeros((pad,), jnp.int32)])
  dst_p = jnp.concatenate([dst, jnp.full((pad,), n_nodes, jnp.int32)])
  dst3 = dst_p.reshape(NS, nch, CE)
  offs1 = (jnp.arange(s1, dtype=jnp.int32) * n_nodes)[:, None]
  gidx1 = (src_p[None, :] + offs1).reshape(s1, NS, nch, CE)
  offs2 = (jnp.arange(s2, dtype=jnp.int32) * n_nodes)[:, None]
  gidx2 = (src_p[None, :] + offs2).reshape(s2, NS, nch, CE)
  zrows = jnp.zeros((n_acc, LW), jnp.float32)

  table1 = x.reshape(n_nodes, s1, LW).transpose(1, 0, 2).reshape(s1 * n_nodes, LW)
  agg1_sl = _segment_sum_sc(s1, n_nodes, nch)(table1, gidx1, dst3, zrows)
  agg1 = agg1_sl[:, :n_nodes, :].transpose(1, 0, 2).reshape(n_nodes, d_in)
  p = _mlp_tc(x, agg1, W1, b1, W2, bm=1000)

  table2 = p.reshape(n_nodes, s2, LW).transpose(1, 0, 2).reshape(s2 * n_nodes, LW)
  agg2_sl = _segment_sum_sc(s2, n_nodes, nch)(table2, gidx2, dst3, zrows)
  agg2 = agg2_sl[:, :n_nodes, :].transpose(1, 0, 2).reshape(n_nodes, d_out)
  out = _add_norm_tc(p, agg2, b2, bm=1000)
  return out
